# baseline (speedup 1.0000x reference)
import functools
import math

import jax
import jax.numpy as jnp
from jax import lax
from jax.experimental import pallas as pl
from jax.experimental.pallas import tpu as pltpu
from jax.experimental.pallas import tpu_sc as plsc

_D = 64
_N = 4096 * 200
_NC = 2
_NS = 16
_NW = _NC * _NS
_PER_W = _N // _NW
_SUB = 128
_K = 4
_C = _SUB * _K
_SROWS = 8
_NSUP = _PER_W // (_SROWS * _SUB)
_LANES = 16
_SCALE = math.sqrt(_D)


def _emb_body(table, idx2, out, idx_v, rows_v, sem0, sem1):
    wid = lax.axis_index("s") * _NC + lax.axis_index("c")
    row_base = wid * _PER_W
    irow_base = wid * (_PER_W // _SUB)
    sems = (sem0, sem1)

    def issue(b, q, h):
        for j in range(_K):
            pltpu.async_copy(
                table.at[idx_v.at[q, h * _K + j]],
                rows_v.at[b, pl.ds(j * _SUB, _SUB)],
                sems[b],
            )

    pltpu.sync_copy(idx2.at[pl.ds(irow_base, _SROWS)], idx_v.at[0])
    issue(0, jnp.int32(0), 0)
    issue(1, jnp.int32(0), 1)
    pltpu.sync_copy(idx2.at[pl.ds(irow_base + _SROWS, _SROWS)], idx_v.at[1])

    @pl.loop(0, _NSUP)
    def _sup(s):
        sp = lax.rem(s, 2)
        spn = lax.rem(s + 1, 2)
        for h in range(2):
            b = h
            for j in range(_K):
                pltpu.make_async_copy(
                    table.at[idx_v.at[sp, h * _K + j]],
                    rows_v.at[b, pl.ds(j * _SUB, _SUB)],
                    sems[b],
                ).wait()

            @plsc.parallel_loop(0, _C, 1, unroll=8)
            def _scale(r):
                for j in range(_D // _LANES):
                    sl = pl.ds(j * _LANES, _LANES)
                    rows_v[b, r, sl] = rows_v[b, r, sl] * _SCALE

            pltpu.sync_copy(
                rows_v.at[b],
                out.at[pl.ds(row_base + (2 * s + h) * _C, _C)],
            )

            @pl.when(s + 1 < _NSUP)
            def _():
                issue(b, spn, h)

        @pl.when(s + 2 < _NSUP)
        def _():
            pltpu.sync_copy(
                idx2.at[pl.ds(irow_base + (s + 2) * _SROWS, _SROWS)],
                idx_v.at[sp],
            )


@functools.partial(
    pl.kernel,
    out_type=jax.ShapeDtypeStruct((_N, _D), jnp.float32),
    mesh=plsc.VectorSubcoreMesh(core_axis_name="c", subcore_axis_name="s"),
    scratch_types=[
        pltpu.VMEM((2, _SROWS, _SUB), jnp.int32),
        pltpu.VMEM((2, _C, _D), jnp.float32),
        pltpu.SemaphoreType.DMA,
        pltpu.SemaphoreType.DMA,
    ],
    compiler_params=pltpu.CompilerParams(use_tc_tiling_on_sc=False),
)
def _emb(table, idx2, out, idx_v, rows_v, sem0, sem1):
    _emb_body(table, idx2, out, idx_v, rows_v, sem0, sem1)


def kernel(x, embedding_weight):
    idx2 = x.astype(jnp.int32).reshape(_N // _SUB, _SUB)
    out = _emb(embedding_weight, idx2)
    return out.reshape(x.shape[0], x.shape[1], _D)
